# baseline (speedup 1.0000x reference)
import functools

import numpy as np

import jax
import jax.numpy as jnp
from jax import lax
from jax.experimental import pallas as pl
from jax.experimental.pallas import tpu as pltpu
from jax.experimental.pallas import tpu_sc as plsc

EPS = 1e-3
NC = 2
NS = 16
LANES = 16


def _interleave_perm(d):
    perm = np.zeros(d, dtype=np.int32)
    for b in range(d // 32):
        for i in range(16):
            perm[32 * b + 2 * i] = 32 * b + i
            perm[32 * b + 2 * i + 1] = 32 * b + 16 + i
    return perm


def _k1_body(x_ref, w1_ref, w2_ref, p_ref, q_ref):
    x = x_ref[...]
    p_ref[...] = jnp.dot(x, w1_ref[...], preferred_element_type=jnp.float32)
    q_ref[...] = jnp.dot(x, w2_ref[...], preferred_element_type=jnp.float32)


def _make_pq(x, w1, w2):
    n, d = x.shape
    return pl.pallas_call(
        _k1_body,
        out_shape=(
            jax.ShapeDtypeStruct((n, d), jnp.float32),
            jax.ShapeDtypeStruct((n, d), jnp.float32),
        ),
    )(x, w1, w2)


def _sc_gather_sum(p, q, idx1, idx0):
    n, d = p.shape
    e = idx1.shape[0]
    nw = NC * NS
    per = e // nw
    assert per * nw == e
    chunk = 80
    nchunks = per // chunk
    assert nchunks * chunk == per
    assert nchunks % 2 == 1

    mesh = plsc.VectorSubcoreMesh(core_axis_name="c", subcore_axis_name="s")

    @functools.partial(
        pl.kernel,
        out_type=jax.ShapeDtypeStruct((e, d), jnp.float32),
        mesh=mesh,
        scratch_types=[
            pltpu.VMEM((per,), jnp.int32),
            pltpu.VMEM((per,), jnp.int32),
            pltpu.VMEM((chunk, d), jnp.float32),
            pltpu.VMEM((chunk, d), jnp.float32),
            pltpu.VMEM((chunk, d), jnp.float32),
            pltpu.VMEM((chunk, d), jnp.float32),
            pltpu.SemaphoreType.DMA,
            pltpu.SemaphoreType.DMA,
            pltpu.SemaphoreType.DMA,
            pltpu.SemaphoreType.DMA,
        ],
    )
    def k2(p_hbm, q_hbm, i1_hbm, i0_hbm, g_hbm,
           i1_v, i0_v, rp_a, rp_b, rq_a, rq_b, sp_a, sp_b, sq_a, sq_b):
        wid = lax.axis_index("s") * NC + lax.axis_index("c")
        base = wid * per

        pltpu.sync_copy(i1_hbm.at[pl.ds(base, per)], i1_v)
        pltpu.sync_copy(i0_hbm.at[pl.ds(base, per)], i0_v)

        def issue(k, rp, rq, sp, sq):
            sl = pl.ds(k * chunk, chunk)
            pltpu.async_copy(p_hbm.at[i1_v.at[sl]], rp, sp)
            pltpu.async_copy(q_hbm.at[i0_v.at[sl]], rq, sq)

        def process(k, rp, rq, sp, sq):
            sl = pl.ds(k * chunk, chunk)
            pltpu.make_async_copy(p_hbm.at[i1_v.at[sl]], rp, sp).wait()
            pltpu.make_async_copy(q_hbm.at[i0_v.at[sl]], rq, sq).wait()

            @plsc.parallel_loop(0, chunk, unroll=2)
            def _row(r):
                for j in range(d // LANES):
                    fsl = pl.ds(j * LANES, LANES)
                    rp[r, fsl] = rp[r, fsl] + rq[r, fsl]

            pltpu.sync_copy(rp, g_hbm.at[pl.ds(base + k * chunk, chunk)])

        issue(0, rp_a, rq_a, sp_a, sq_a)

        def pair(kk, carry):
            k0 = 2 * kk
            issue(k0 + 1, rp_b, rq_b, sp_b, sq_b)
            process(k0, rp_a, rq_a, sp_a, sq_a)
            issue(k0 + 2, rp_a, rq_a, sp_a, sq_a)
            process(k0 + 1, rp_b, rq_b, sp_b, sq_b)
            return carry

        lax.fori_loop(0, (nchunks - 1) // 2, pair, 0)
        process(nchunks - 1, rp_a, rq_a, sp_a, sq_a)

    return k2(p, q, idx1, idx0)


def _k3a_body(nsteps, etotal, d, g_ref, ea_ref, w3_ref, be_ref,
              gam_ref, bet_ref, s_ref, ss_ref, a_ref, c_ref):
    i = pl.program_id(0)
    h = (g_ref[...]
         + jnp.dot(ea_ref[...], w3_ref[...], preferred_element_type=jnp.float32)
         + be_ref[...])
    h = jnp.maximum(h, 0.0)

    @pl.when(i == 0)
    def _():
        s_ref[...] = jnp.zeros_like(s_ref)
        ss_ref[...] = jnp.zeros_like(ss_ref)

    s_ref[...] += jnp.sum(h, axis=0, keepdims=True)
    ss_ref[...] += jnp.sum(h * h, axis=0, keepdims=True)

    @pl.when(i == nsteps - 1)
    def _():
        s = s_ref[...]
        ss = ss_ref[...]
        sf = jnp.zeros((1, d), jnp.float32)
        ssf = jnp.zeros((1, d), jnp.float32)
        for t in range(s.shape[1] // d):
            sf = sf + s[:, t * d:(t + 1) * d]
            ssf = ssf + ss[:, t * d:(t + 1) * d]
        mean = sf / etotal
        var = ssf / etotal - mean * mean
        a = gam_ref[...] * lax.rsqrt(var + EPS)
        a_ref[...] = a
        c_ref[...] = bet_ref[...] - mean * a


def _edge_stats(g8, ea8, w3big, b_e8, gamma_e, beta_e, d):
    e8, d8 = g8.shape
    de8 = ea8.shape[1]
    blk = 400
    nsteps = e8 // blk
    assert nsteps * blk == e8
    body = functools.partial(_k3a_body, nsteps, float(e8 * (d8 // d)), d)
    rvec = pl.BlockSpec((1, d), lambda i: (0, 0))
    rvec8 = pl.BlockSpec((1, d8), lambda i: (0, 0))
    return pl.pallas_call(
        body,
        grid=(nsteps,),
        in_specs=[
            pl.BlockSpec((blk, d8), lambda i: (i, 0)),
            pl.BlockSpec((blk, de8), lambda i: (i, 0)),
            pl.BlockSpec((de8, d8), lambda i: (0, 0)),
            rvec8, rvec, rvec,
        ],
        out_specs=[rvec8, rvec8, rvec, rvec],
        out_shape=[
            jax.ShapeDtypeStruct((1, d8), jnp.float32),
            jax.ShapeDtypeStruct((1, d8), jnp.float32),
            jax.ShapeDtypeStruct((1, d), jnp.float32),
            jax.ShapeDtypeStruct((1, d), jnp.float32),
        ],
    )(g8, ea8, w3big, b_e8, gamma_e.reshape(1, d), beta_e.reshape(1, d))


def _k3b_body(g_ref, ea_ref, w3_ref, be_ref, a_ref, c_ref, eu_ref):
    h = (g_ref[...]
         + jnp.dot(ea_ref[...], w3_ref[...], preferred_element_type=jnp.float32)
         + be_ref[...])
    h = jnp.maximum(h, 0.0)
    eu_ref[...] = jnp.maximum(a_ref[...] * h + c_ref[...], 0.0)


def _edge_apply(g8, ea8, w3big, b_e8, a8, c8):
    e8, d8 = g8.shape
    de8 = ea8.shape[1]
    blk = 400
    nsteps = e8 // blk
    rvec8 = pl.BlockSpec((1, d8), lambda i: (0, 0))
    return pl.pallas_call(
        _k3b_body,
        grid=(nsteps,),
        in_specs=[
            pl.BlockSpec((blk, d8), lambda i: (i, 0)),
            pl.BlockSpec((blk, de8), lambda i: (i, 0)),
            pl.BlockSpec((de8, d8), lambda i: (0, 0)),
            rvec8, rvec8, rvec8,
        ],
        out_specs=pl.BlockSpec((blk, d8), lambda i: (i, 0)),
        out_shape=jax.ShapeDtypeStruct((e8, d8), jnp.float32),
    )(g8, ea8, w3big, b_e8, a8, c8)


def _sc_scatter(eu, rev0, n):
    e, d = eu.shape
    nw = NC * NS
    per = e // nw
    chunk = 80
    nchunks = per // chunk
    assert nchunks * chunk == per
    zblk = 128
    rows_per_tile = ((n + NS - 1) // NS + zblk - 1) // zblk * zblk
    npad = NS * rows_per_tile
    nz = rows_per_tile // zblk

    mesh = plsc.VectorSubcoreMesh(core_axis_name="c", subcore_axis_name="s")

    @functools.partial(
        pl.kernel,
        out_type=jax.ShapeDtypeStruct((NC, npad, d), jnp.float32),
        mesh=mesh,
        scratch_types=[
            pltpu.VMEM((chunk,), jnp.int32),
            pltpu.VMEM((chunk,), jnp.int32),
            pltpu.VMEM((chunk, d), jnp.float32),
            pltpu.VMEM((chunk, d), jnp.float32),
            pltpu.VMEM((zblk, d), jnp.float32),
            pltpu.VMEM_SHARED((npad, d), jnp.float32),
            pltpu.SemaphoreType.DMA,
            pltpu.SemaphoreType.DMA,
            pltpu.SemaphoreType.DMA,
            pltpu.SemaphoreType.DMA,
        ],
    )
    def k4(eu_hbm, rev_hbm, agg_hbm,
           idx_a, idx_b, h_a, h_b, z_v, agg_sh, sh_a, sh_b, si_a, si_b):
        cid = lax.axis_index("c")
        sid = lax.axis_index("s")
        wid = sid * NC + cid
        base = wid * per

        @plsc.parallel_loop(0, zblk, unroll=2)
        def _zrow(r):
            for j in range(d // LANES):
                z_v[r, pl.ds(j * LANES, LANES)] = jnp.zeros((LANES,), jnp.float32)

        for t in range(nz):
            pltpu.sync_copy(z_v, agg_sh.at[pl.ds(sid * rows_per_tile + t * zblk, zblk)])
        plsc.subcore_barrier()

        def issue(k, h_v, idx_v, sh, si):
            off = base + k * chunk
            pltpu.async_copy(eu_hbm.at[pl.ds(off, chunk)], h_v, sh)
            pltpu.async_copy(rev_hbm.at[pl.ds(off, chunk)], idx_v, si)

        def process(k, h_v, idx_v, sh, si):
            off = base + k * chunk
            pltpu.make_async_copy(eu_hbm.at[pl.ds(off, chunk)], h_v, sh).wait()
            pltpu.make_async_copy(rev_hbm.at[pl.ds(off, chunk)], idx_v, si).wait()
            pltpu.sync_copy(h_v, agg_sh.at[idx_v], add=True)

        issue(0, h_a, idx_a, sh_a, si_a)

        def pair(kk, carry):
            k0 = 2 * kk
            issue(k0 + 1, h_b, idx_b, sh_b, si_b)
            process(k0, h_a, idx_a, sh_a, si_a)
            issue(k0 + 2, h_a, idx_a, sh_a, si_a)
            process(k0 + 1, h_b, idx_b, sh_b, si_b)
            return carry

        assert nchunks % 2 == 1
        lax.fori_loop(0, (nchunks - 1) // 2, pair, 0)
        process(nchunks - 1, h_a, idx_a, sh_a, si_a)
        plsc.subcore_barrier()

        for t in range(nz):
            r0 = sid * rows_per_tile + t * zblk
            pltpu.sync_copy(agg_sh.at[pl.ds(r0, zblk)], z_v)
            pltpu.sync_copy(z_v, agg_hbm.at[cid, pl.ds(r0, zblk)])

    return k4(eu, rev0)


def _k5_body(x_ref, agg_ref, w1a_ref, w1b_ref, b1_ref, g1_ref, be1_ref,
             w2_ref, b2_ref, g2_ref, be2_ref, gn_ref, ben_ref,
             wn_ref, bn_ref, out_ref):
    n = x_ref.shape[0]
    ones = jnp.ones((1, n), jnp.float32)

    def bn(x, g, b):
        m = jnp.dot(ones, x, preferred_element_type=jnp.float32) / n
        xc = x - m
        v = jnp.dot(ones, xc * xc, preferred_element_type=jnp.float32) / n
        return g * xc * lax.rsqrt(v + EPS) + b

    agg = agg_ref[0] + agg_ref[1]
    h = (jnp.dot(x_ref[...], w1a_ref[...], preferred_element_type=jnp.float32)
         + jnp.dot(agg, w1b_ref[...], preferred_element_type=jnp.float32)
         + b1_ref[...])
    h = jnp.maximum(bn(h, g1_ref[...], be1_ref[...]), 0.0)
    h = jnp.dot(h, w2_ref[...], preferred_element_type=jnp.float32) + b2_ref[...]
    h = jnp.maximum(bn(h, g2_ref[...], be2_ref[...]), 0.0)
    h = bn(h, gn_ref[...], ben_ref[...])
    h = jnp.dot(h, wn_ref[...], preferred_element_type=jnp.float32) + bn_ref[...]
    out_ref[...] = jnp.maximum(h, 0.0)


def _node_mlp(x, aggp, w1a, w1b, b1, g1, be1, w2, b2, g2, be2, gn, ben, wn, bn):
    n, d = x.shape
    r = lambda v: v.reshape(1, d)
    return pl.pallas_call(
        _k5_body,
        out_shape=jax.ShapeDtypeStruct((n, d), jnp.float32),
    )(x, aggp, w1a, w1b, r(b1), r(g1), r(be1), w2, r(b2), r(g2), r(be2),
      r(gn), r(ben), wn, r(bn))


def kernel(node_attributes, edge_attributes, edge_indices, edge_indices_reverse,
           W_e, b_e, gamma_e, beta_e, W_g1, b_g1, gamma_g1, beta_g1,
           W_g2, b_g2, gamma_g2, beta_g2, gamma_n, beta_n, W_nu, b_nu):
    n, d = node_attributes.shape
    idx1 = edge_indices[:, 1].astype(jnp.int32)
    idx0 = edge_indices[:, 0].astype(jnp.int32)
    rev0 = edge_indices_reverse[:, 0].astype(jnp.int32)

    p, q = _make_pq(node_attributes, W_e[:d], W_e[d:2 * d])
    g = _sc_gather_sum(p, q, idx1, idx0)
    e = g.shape[0]
    pack = 8
    g8 = g.reshape(e // pack, pack * d)
    ea8 = edge_attributes.reshape(e // pack, pack * edge_attributes.shape[1])
    w3 = W_e[2 * d:]
    w3big = jnp.kron(jnp.eye(pack, dtype=jnp.float32), w3)
    b_e8 = jnp.tile(b_e, pack).reshape(1, pack * d)
    _, _, a, c = _edge_stats(g8, ea8, w3big, b_e8, gamma_e, beta_e, d)
    a8 = jnp.tile(a.reshape(d), pack).reshape(1, pack * d)
    c8 = jnp.tile(c.reshape(d), pack).reshape(1, pack * d)
    eu8 = _edge_apply(g8, ea8, w3big, b_e8, a8, c8)
    eu = eu8.reshape(e, d)
    aggp = _sc_scatter(eu, rev0, n)
    aggp = aggp[:, :n]
    node_final = _node_mlp(node_attributes, aggp,
                           W_g1[:d], W_g1[d:], b_g1, gamma_g1, beta_g1,
                           W_g2, b_g2, gamma_g2, beta_g2,
                           gamma_n, beta_n, W_nu, b_nu)
    return (node_final, eu)


# speedup vs baseline: 1.5508x; 1.5508x over previous
import functools

import numpy as np

import jax
import jax.numpy as jnp
from jax import lax
from jax.experimental import pallas as pl
from jax.experimental.pallas import tpu as pltpu
from jax.experimental.pallas import tpu_sc as plsc

EPS = 1e-3
NC = 2
NS = 16
LANES = 16


def _interleave_perm(d):
    perm = np.zeros(d, dtype=np.int32)
    for b in range(d // 32):
        for i in range(16):
            perm[32 * b + 2 * i] = 32 * b + i
            perm[32 * b + 2 * i + 1] = 32 * b + 16 + i
    return perm


def _k1_body(x_ref, w1_ref, w2_ref, p_ref, q_ref):
    x = x_ref[...]
    p_ref[...] = jnp.dot(x, w1_ref[...], preferred_element_type=jnp.float32)
    q_ref[...] = jnp.dot(x, w2_ref[...], preferred_element_type=jnp.float32)


def _make_pq(x, w1, w2):
    n, d = x.shape
    return pl.pallas_call(
        _k1_body,
        out_shape=(
            jax.ShapeDtypeStruct((n, d), jnp.float32),
            jax.ShapeDtypeStruct((n, d), jnp.float32),
        ),
    )(x, w1, w2)


def _sc_gather_sum(p, q, idx1, idx0):
    n, d = p.shape
    e = idx1.shape[0]
    nw = NC * NS
    per = e // nw
    assert per * nw == e
    chunk = 80
    nchunks = per // chunk
    assert nchunks * chunk == per
    assert nchunks % 2 == 1

    mesh = plsc.VectorSubcoreMesh(core_axis_name="c", subcore_axis_name="s")

    @functools.partial(
        pl.kernel,
        out_type=jax.ShapeDtypeStruct((e, d), jnp.float32),
        mesh=mesh,
        scratch_types=[
            pltpu.VMEM((per,), jnp.int32),
            pltpu.VMEM((per,), jnp.int32),
            pltpu.VMEM((chunk, d), jnp.float32),
            pltpu.VMEM((chunk, d), jnp.float32),
            pltpu.VMEM((chunk, d), jnp.float32),
            pltpu.VMEM((chunk, d), jnp.float32),
            pltpu.SemaphoreType.DMA,
            pltpu.SemaphoreType.DMA,
            pltpu.SemaphoreType.DMA,
            pltpu.SemaphoreType.DMA,
        ],
    )
    def k2(p_hbm, q_hbm, i1_hbm, i0_hbm, g_hbm,
           i1_v, i0_v, rp_a, rp_b, rq_a, rq_b, sp_a, sp_b, sq_a, sq_b):
        wid = lax.axis_index("s") * NC + lax.axis_index("c")
        base = wid * per

        pltpu.sync_copy(i1_hbm.at[pl.ds(base, per)], i1_v)
        pltpu.sync_copy(i0_hbm.at[pl.ds(base, per)], i0_v)

        def issue(k, rp, rq, sp, sq):
            sl = pl.ds(k * chunk, chunk)
            pltpu.async_copy(p_hbm.at[i1_v.at[sl]], rp, sp)
            pltpu.async_copy(q_hbm.at[i0_v.at[sl]], rq, sq)

        def process(k, rp, rq, sp, sq):
            sl = pl.ds(k * chunk, chunk)
            pltpu.make_async_copy(p_hbm.at[i1_v.at[sl]], rp, sp).wait()
            pltpu.make_async_copy(q_hbm.at[i0_v.at[sl]], rq, sq).wait()

            @plsc.parallel_loop(0, chunk, unroll=2)
            def _row(r):
                for j in range(d // LANES):
                    fsl = pl.ds(j * LANES, LANES)
                    rp[r, fsl] = rp[r, fsl] + rq[r, fsl]

            pltpu.sync_copy(rp, g_hbm.at[pl.ds(base + k * chunk, chunk)])

        issue(0, rp_a, rq_a, sp_a, sq_a)

        def pair(kk, carry):
            k0 = 2 * kk
            issue(k0 + 1, rp_b, rq_b, sp_b, sq_b)
            process(k0, rp_a, rq_a, sp_a, sq_a)
            issue(k0 + 2, rp_a, rq_a, sp_a, sq_a)
            process(k0 + 1, rp_b, rq_b, sp_b, sq_b)
            return carry

        lax.fori_loop(0, (nchunks - 1) // 2, pair, 0)
        process(nchunks - 1, rp_a, rq_a, sp_a, sq_a)

    return k2(p, q, idx1, idx0)


def _k3a_body(nsteps, etotal, g_ref, eat_ref, w3_ref, be_ref,
              gam_ref, bet_ref, s_ref, ss_ref, a_ref, c_ref):
    i = pl.program_id(0)
    ea_term = lax.dot_general(eat_ref[...], w3_ref[...],
                              (((0,), (0,)), ((), ())),
                              preferred_element_type=jnp.float32)
    h = g_ref[...] + ea_term + be_ref[...]
    h = jnp.maximum(h, 0.0)

    @pl.when(i == 0)
    def _():
        s_ref[...] = jnp.zeros_like(s_ref)
        ss_ref[...] = jnp.zeros_like(ss_ref)

    s_ref[...] += jnp.sum(h, axis=0, keepdims=True)
    ss_ref[...] += jnp.sum(h * h, axis=0, keepdims=True)

    @pl.when(i == nsteps - 1)
    def _():
        mean = s_ref[...] / etotal
        var = ss_ref[...] / etotal - mean * mean
        a = gam_ref[...] * lax.rsqrt(var + EPS)
        a_ref[...] = a
        c_ref[...] = bet_ref[...] - mean * a


def _edge_stats(g, eat, w3, b_e, gamma_e, beta_e):
    e, d = g.shape
    de = eat.shape[0]
    blk = 3200
    nsteps = e // blk
    assert nsteps * blk == e
    body = functools.partial(_k3a_body, nsteps, float(e))
    rvec = pl.BlockSpec((1, d), lambda i: (0, 0))
    return pl.pallas_call(
        body,
        grid=(nsteps,),
        in_specs=[
            pl.BlockSpec((blk, d), lambda i: (i, 0)),
            pl.BlockSpec((de, blk), lambda i: (0, i)),
            pl.BlockSpec((de, d), lambda i: (0, 0)),
            rvec, rvec, rvec,
        ],
        out_specs=[rvec, rvec, rvec, rvec],
        out_shape=[jax.ShapeDtypeStruct((1, d), jnp.float32)] * 4,
    )(g, eat, w3, b_e.reshape(1, d), gamma_e.reshape(1, d),
      beta_e.reshape(1, d))


def _k3b_body(g_ref, eat_ref, w3_ref, be_ref, a_ref, c_ref, eu_ref):
    ea_term = lax.dot_general(eat_ref[...], w3_ref[...],
                              (((0,), (0,)), ((), ())),
                              preferred_element_type=jnp.float32)
    h = g_ref[...] + ea_term + be_ref[...]
    h = jnp.maximum(h, 0.0)
    eu_ref[...] = jnp.maximum(a_ref[...] * h + c_ref[...], 0.0)


def _edge_apply(g, eat, w3, b_e, a, c):
    e, d = g.shape
    de = eat.shape[0]
    blk = 3200
    nsteps = e // blk
    rvec = pl.BlockSpec((1, d), lambda i: (0, 0))
    return pl.pallas_call(
        _k3b_body,
        grid=(nsteps,),
        in_specs=[
            pl.BlockSpec((blk, d), lambda i: (i, 0)),
            pl.BlockSpec((de, blk), lambda i: (0, i)),
            pl.BlockSpec((de, d), lambda i: (0, 0)),
            rvec, rvec, rvec,
        ],
        out_specs=pl.BlockSpec((blk, d), lambda i: (i, 0)),
        out_shape=jax.ShapeDtypeStruct((e, d), jnp.float32),
    )(g, eat, w3, b_e.reshape(1, d), a, c)


def _sc_scatter(eu, rev0, n):
    e, d = eu.shape
    nw = NC * NS
    per = e // nw
    chunk = 80
    nchunks = per // chunk
    assert nchunks * chunk == per
    zblk = 128
    rows_per_tile = ((n + NS - 1) // NS + zblk - 1) // zblk * zblk
    npad = NS * rows_per_tile
    nz = rows_per_tile // zblk

    mesh = plsc.VectorSubcoreMesh(core_axis_name="c", subcore_axis_name="s")

    @functools.partial(
        pl.kernel,
        out_type=jax.ShapeDtypeStruct((NC, npad, d), jnp.float32),
        mesh=mesh,
        scratch_types=[
            pltpu.VMEM((chunk,), jnp.int32),
            pltpu.VMEM((chunk,), jnp.int32),
            pltpu.VMEM((chunk, d), jnp.float32),
            pltpu.VMEM((chunk, d), jnp.float32),
            pltpu.VMEM((zblk, d), jnp.float32),
            pltpu.VMEM_SHARED((npad, d), jnp.float32),
            pltpu.SemaphoreType.DMA,
            pltpu.SemaphoreType.DMA,
            pltpu.SemaphoreType.DMA,
            pltpu.SemaphoreType.DMA,
        ],
    )
    def k4(eu_hbm, rev_hbm, agg_hbm,
           idx_a, idx_b, h_a, h_b, z_v, agg_sh, sh_a, sh_b, si_a, si_b):
        cid = lax.axis_index("c")
        sid = lax.axis_index("s")
        wid = sid * NC + cid
        base = wid * per

        @plsc.parallel_loop(0, zblk, unroll=2)
        def _zrow(r):
            for j in range(d // LANES):
                z_v[r, pl.ds(j * LANES, LANES)] = jnp.zeros((LANES,), jnp.float32)

        for t in range(nz):
            pltpu.sync_copy(z_v, agg_sh.at[pl.ds(sid * rows_per_tile + t * zblk, zblk)])
        plsc.subcore_barrier()

        def issue(k, h_v, idx_v, sh, si):
            off = base + k * chunk
            pltpu.async_copy(eu_hbm.at[pl.ds(off, chunk)], h_v, sh)
            pltpu.async_copy(rev_hbm.at[pl.ds(off, chunk)], idx_v, si)

        def process(k, h_v, idx_v, sh, si):
            off = base + k * chunk
            pltpu.make_async_copy(eu_hbm.at[pl.ds(off, chunk)], h_v, sh).wait()
            pltpu.make_async_copy(rev_hbm.at[pl.ds(off, chunk)], idx_v, si).wait()
            pltpu.sync_copy(h_v, agg_sh.at[idx_v], add=True)

        issue(0, h_a, idx_a, sh_a, si_a)

        def pair(kk, carry):
            k0 = 2 * kk
            issue(k0 + 1, h_b, idx_b, sh_b, si_b)
            process(k0, h_a, idx_a, sh_a, si_a)
            issue(k0 + 2, h_a, idx_a, sh_a, si_a)
            process(k0 + 1, h_b, idx_b, sh_b, si_b)
            return carry

        assert nchunks % 2 == 1
        lax.fori_loop(0, (nchunks - 1) // 2, pair, 0)
        process(nchunks - 1, h_a, idx_a, sh_a, si_a)
        plsc.subcore_barrier()

        for t in range(nz):
            r0 = sid * rows_per_tile + t * zblk
            pltpu.sync_copy(agg_sh.at[pl.ds(r0, zblk)], z_v)
            pltpu.sync_copy(z_v, agg_hbm.at[cid, pl.ds(r0, zblk)])

    return k4(eu, rev0)


def _k5_body(x_ref, agg_ref, w1a_ref, w1b_ref, b1_ref, g1_ref, be1_ref,
             w2_ref, b2_ref, g2_ref, be2_ref, gn_ref, ben_ref,
             wn_ref, bn_ref, out_ref):
    n = x_ref.shape[0]
    ones = jnp.ones((1, n), jnp.float32)

    def bn(x, g, b):
        m = jnp.dot(ones, x, preferred_element_type=jnp.float32) / n
        xc = x - m
        v = jnp.dot(ones, xc * xc, preferred_element_type=jnp.float32) / n
        return g * xc * lax.rsqrt(v + EPS) + b

    agg = agg_ref[0] + agg_ref[1]
    h = (jnp.dot(x_ref[...], w1a_ref[...], preferred_element_type=jnp.float32)
         + jnp.dot(agg, w1b_ref[...], preferred_element_type=jnp.float32)
         + b1_ref[...])
    h = jnp.maximum(bn(h, g1_ref[...], be1_ref[...]), 0.0)
    h = jnp.dot(h, w2_ref[...], preferred_element_type=jnp.float32) + b2_ref[...]
    h = jnp.maximum(bn(h, g2_ref[...], be2_ref[...]), 0.0)
    h = bn(h, gn_ref[...], ben_ref[...])
    h = jnp.dot(h, wn_ref[...], preferred_element_type=jnp.float32) + bn_ref[...]
    out_ref[...] = jnp.maximum(h, 0.0)


def _node_mlp(x, aggp, w1a, w1b, b1, g1, be1, w2, b2, g2, be2, gn, ben, wn, bn):
    n, d = x.shape
    r = lambda v: v.reshape(1, d)
    return pl.pallas_call(
        _k5_body,
        out_shape=jax.ShapeDtypeStruct((n, d), jnp.float32),
    )(x, aggp, w1a, w1b, r(b1), r(g1), r(be1), w2, r(b2), r(g2), r(be2),
      r(gn), r(ben), wn, r(bn))


def kernel(node_attributes, edge_attributes, edge_indices, edge_indices_reverse,
           W_e, b_e, gamma_e, beta_e, W_g1, b_g1, gamma_g1, beta_g1,
           W_g2, b_g2, gamma_g2, beta_g2, gamma_n, beta_n, W_nu, b_nu):
    n, d = node_attributes.shape
    idx1 = edge_indices[:, 1].astype(jnp.int32)
    idx0 = edge_indices[:, 0].astype(jnp.int32)
    rev0 = edge_indices_reverse[:, 0].astype(jnp.int32)

    p, q = _make_pq(node_attributes, W_e[:d], W_e[d:2 * d])
    g = _sc_gather_sum(p, q, idx1, idx0)
    eat = edge_attributes.T
    w3 = W_e[2 * d:]
    _, _, a, c = _edge_stats(g, eat, w3, b_e, gamma_e, beta_e)
    eu = _edge_apply(g, eat, w3, b_e, a, c)
    aggp = _sc_scatter(eu, rev0, n)
    aggp = aggp[:, :n]
    node_final = _node_mlp(node_attributes, aggp,
                           W_g1[:d], W_g1[d:], b_g1, gamma_g1, beta_g1,
                           W_g2, b_g2, gamma_g2, beta_g2,
                           gamma_n, beta_n, W_nu, b_nu)
    return (node_final, eu)


# speedup vs baseline: 1.5788x; 1.0180x over previous
import functools

import numpy as np

import jax
import jax.numpy as jnp
from jax import lax
from jax.experimental import pallas as pl
from jax.experimental.pallas import tpu as pltpu
from jax.experimental.pallas import tpu_sc as plsc

EPS = 1e-3
NC = 2
NS = 16
LANES = 16


def _interleave_perm(d):
    perm = np.zeros(d, dtype=np.int32)
    for b in range(d // 32):
        for i in range(16):
            perm[32 * b + 2 * i] = 32 * b + i
            perm[32 * b + 2 * i + 1] = 32 * b + 16 + i
    return perm


def _k1_body(x_ref, w1_ref, w2_ref, p_ref, q_ref):
    x = x_ref[...]
    p_ref[...] = jnp.dot(x, w1_ref[...], preferred_element_type=jnp.float32)
    q_ref[...] = jnp.dot(x, w2_ref[...], preferred_element_type=jnp.float32)


def _make_pq(x, w1, w2):
    n, d = x.shape
    return pl.pallas_call(
        _k1_body,
        out_shape=(
            jax.ShapeDtypeStruct((n, d), jnp.float32),
            jax.ShapeDtypeStruct((n, d), jnp.float32),
        ),
    )(x, w1, w2)


def _sc_gather_sum(p, q, idx1, idx0):
    n, d = p.shape
    e = idx1.shape[0]
    nw = NC * NS
    per = e // nw
    assert per * nw == e
    chunk = 80
    nchunks = per // chunk
    assert nchunks * chunk == per
    assert nchunks % 3 == 2

    mesh = plsc.VectorSubcoreMesh(core_axis_name="c", subcore_axis_name="s")

    buf_scratch = []
    for _ in range(3):
        buf_scratch += [
            pltpu.VMEM((chunk, d), jnp.float32),
            pltpu.VMEM((chunk, d), jnp.float32),
            pltpu.SemaphoreType.DMA,
            pltpu.SemaphoreType.DMA,
            pltpu.SemaphoreType.DMA,
        ]

    @functools.partial(
        pl.kernel,
        out_type=jax.ShapeDtypeStruct((e, d), jnp.float32),
        mesh=mesh,
        scratch_types=[
            pltpu.VMEM((per,), jnp.int32),
            pltpu.VMEM((per,), jnp.int32),
        ] + buf_scratch,
    )
    def k2(p_hbm, q_hbm, i1_hbm, i0_hbm, g_hbm, i1_v, i0_v, *bufrefs):
        wid = lax.axis_index("s") * NC + lax.axis_index("c")
        base = wid * per
        bufs = [tuple(bufrefs[5 * t:5 * t + 5]) for t in range(3)]

        pltpu.sync_copy(i1_hbm.at[pl.ds(base, per)], i1_v)
        pltpu.sync_copy(i0_hbm.at[pl.ds(base, per)], i0_v)

        def issue(k, buf):
            rp, rq, sp, sq, _ = buf
            sl = pl.ds(k * chunk, chunk)
            pltpu.async_copy(p_hbm.at[i1_v.at[sl]], rp, sp)
            pltpu.async_copy(q_hbm.at[i0_v.at[sl]], rq, sq)

        def drain_write(k, buf):
            rp, _, _, _, ws = buf
            pltpu.make_async_copy(
                rp, g_hbm.at[pl.ds(base + k * chunk, chunk)], ws).wait()

        def process_add(k, buf):
            rp, rq, sp, sq, _ = buf
            sl = pl.ds(k * chunk, chunk)
            pltpu.make_async_copy(p_hbm.at[i1_v.at[sl]], rp, sp).wait()
            pltpu.make_async_copy(q_hbm.at[i0_v.at[sl]], rq, sq).wait()

            @plsc.parallel_loop(0, chunk, unroll=2)
            def _row(r):
                for j in range(d // LANES):
                    fsl = pl.ds(j * LANES, LANES)
                    rp[r, fsl] = rp[r, fsl] + rq[r, fsl]

        def write_async(k, buf):
            rp, _, _, _, ws = buf
            pltpu.async_copy(rp, g_hbm.at[pl.ds(base + k * chunk, chunk)], ws)

        issue(0, bufs[0])
        issue(1, bufs[1])

        def body3(kk, carry):
            k0 = 3 * kk
            for t in range(3):
                k = k0 + t
                b_cur = bufs[t]
                b_nxt = bufs[(t + 2) % 3]

                @pl.when(k >= 1)
                def _():
                    drain_write(k - 1, b_nxt)

                issue(k + 2, b_nxt)
                process_add(k, b_cur)
                write_async(k, b_cur)
            return carry

        lax.fori_loop(0, (nchunks - 2) // 3, body3, 0)
        for k, bi in ((nchunks - 2, 0), (nchunks - 1, 1)):
            rp = bufs[bi][0]
            process_add(k, bufs[bi])
            pltpu.sync_copy(rp, g_hbm.at[pl.ds(base + k * chunk, chunk)])
        drain_write(nchunks - 3, bufs[2])

    return k2(p, q, idx1, idx0)


def _k3a_body(nsteps, etotal, g_ref, eat_ref, w3_ref, be_ref,
              gam_ref, bet_ref, s_ref, ss_ref, a_ref, c_ref):
    i = pl.program_id(0)
    ea_term = lax.dot_general(eat_ref[...], w3_ref[...],
                              (((0,), (0,)), ((), ())),
                              preferred_element_type=jnp.float32)
    h = g_ref[...] + ea_term + be_ref[...]
    h = jnp.maximum(h, 0.0)

    @pl.when(i == 0)
    def _():
        s_ref[...] = jnp.zeros_like(s_ref)
        ss_ref[...] = jnp.zeros_like(ss_ref)

    s_ref[...] += jnp.sum(h, axis=0, keepdims=True)
    ss_ref[...] += jnp.sum(h * h, axis=0, keepdims=True)

    @pl.when(i == nsteps - 1)
    def _():
        mean = s_ref[...] / etotal
        var = ss_ref[...] / etotal - mean * mean
        a = gam_ref[...] * lax.rsqrt(var + EPS)
        a_ref[...] = a
        c_ref[...] = bet_ref[...] - mean * a


def _edge_stats(g, eat, w3, b_e, gamma_e, beta_e):
    e, d = g.shape
    de = eat.shape[0]
    blk = 3200
    nsteps = e // blk
    assert nsteps * blk == e
    body = functools.partial(_k3a_body, nsteps, float(e))
    rvec = pl.BlockSpec((1, d), lambda i: (0, 0))
    return pl.pallas_call(
        body,
        grid=(nsteps,),
        in_specs=[
            pl.BlockSpec((blk, d), lambda i: (i, 0)),
            pl.BlockSpec((de, blk), lambda i: (0, i)),
            pl.BlockSpec((de, d), lambda i: (0, 0)),
            rvec, rvec, rvec,
        ],
        out_specs=[rvec, rvec, rvec, rvec],
        out_shape=[jax.ShapeDtypeStruct((1, d), jnp.float32)] * 4,
    )(g, eat, w3, b_e.reshape(1, d), gamma_e.reshape(1, d),
      beta_e.reshape(1, d))


def _k3b_body(g_ref, eat_ref, w3_ref, be_ref, a_ref, c_ref, eu_ref):
    ea_term = lax.dot_general(eat_ref[...], w3_ref[...],
                              (((0,), (0,)), ((), ())),
                              preferred_element_type=jnp.float32)
    h = g_ref[...] + ea_term + be_ref[...]
    h = jnp.maximum(h, 0.0)
    eu_ref[...] = jnp.maximum(a_ref[...] * h + c_ref[...], 0.0)


def _edge_apply(g, eat, w3, b_e, a, c):
    e, d = g.shape
    de = eat.shape[0]
    blk = 3200
    nsteps = e // blk
    rvec = pl.BlockSpec((1, d), lambda i: (0, 0))
    return pl.pallas_call(
        _k3b_body,
        grid=(nsteps,),
        in_specs=[
            pl.BlockSpec((blk, d), lambda i: (i, 0)),
            pl.BlockSpec((de, blk), lambda i: (0, i)),
            pl.BlockSpec((de, d), lambda i: (0, 0)),
            rvec, rvec, rvec,
        ],
        out_specs=pl.BlockSpec((blk, d), lambda i: (i, 0)),
        out_shape=jax.ShapeDtypeStruct((e, d), jnp.float32),
    )(g, eat, w3, b_e.reshape(1, d), a, c)


def _sc_scatter(eu, rev0, n):
    e, d = eu.shape
    nw = NC * NS
    per = e // nw
    chunk = 80
    nchunks = per // chunk
    assert nchunks * chunk == per
    zblk = 128
    rows_per_tile = ((n + NS - 1) // NS + zblk - 1) // zblk * zblk
    npad = NS * rows_per_tile
    nz = rows_per_tile // zblk

    mesh = plsc.VectorSubcoreMesh(core_axis_name="c", subcore_axis_name="s")

    @functools.partial(
        pl.kernel,
        out_type=jax.ShapeDtypeStruct((NC, npad, d), jnp.float32),
        mesh=mesh,
        scratch_types=[
            pltpu.VMEM((chunk,), jnp.int32),
            pltpu.VMEM((chunk,), jnp.int32),
            pltpu.VMEM((chunk, d), jnp.float32),
            pltpu.VMEM((chunk, d), jnp.float32),
            pltpu.VMEM((zblk, d), jnp.float32),
            pltpu.VMEM_SHARED((npad, d), jnp.float32),
            pltpu.SemaphoreType.DMA,
            pltpu.SemaphoreType.DMA,
            pltpu.SemaphoreType.DMA,
            pltpu.SemaphoreType.DMA,
        ],
    )
    def k4(eu_hbm, rev_hbm, agg_hbm,
           idx_a, idx_b, h_a, h_b, z_v, agg_sh, sh_a, sh_b, si_a, si_b):
        cid = lax.axis_index("c")
        sid = lax.axis_index("s")
        wid = sid * NC + cid
        base = wid * per

        @plsc.parallel_loop(0, zblk, unroll=2)
        def _zrow(r):
            for j in range(d // LANES):
                z_v[r, pl.ds(j * LANES, LANES)] = jnp.zeros((LANES,), jnp.float32)

        for t in range(nz):
            pltpu.sync_copy(z_v, agg_sh.at[pl.ds(sid * rows_per_tile + t * zblk, zblk)])
        plsc.subcore_barrier()

        def issue(k, h_v, idx_v, sh, si):
            off = base + k * chunk
            pltpu.async_copy(eu_hbm.at[pl.ds(off, chunk)], h_v, sh)
            pltpu.async_copy(rev_hbm.at[pl.ds(off, chunk)], idx_v, si)

        def process(k, h_v, idx_v, sh, si):
            off = base + k * chunk
            pltpu.make_async_copy(eu_hbm.at[pl.ds(off, chunk)], h_v, sh).wait()
            pltpu.make_async_copy(rev_hbm.at[pl.ds(off, chunk)], idx_v, si).wait()
            pltpu.sync_copy(h_v, agg_sh.at[idx_v], add=True)

        issue(0, h_a, idx_a, sh_a, si_a)

        def pair(kk, carry):
            k0 = 2 * kk
            issue(k0 + 1, h_b, idx_b, sh_b, si_b)
            process(k0, h_a, idx_a, sh_a, si_a)
            issue(k0 + 2, h_a, idx_a, sh_a, si_a)
            process(k0 + 1, h_b, idx_b, sh_b, si_b)
            return carry

        assert nchunks % 2 == 1
        lax.fori_loop(0, (nchunks - 1) // 2, pair, 0)
        process(nchunks - 1, h_a, idx_a, sh_a, si_a)
        plsc.subcore_barrier()

        for t in range(nz):
            r0 = sid * rows_per_tile + t * zblk
            pltpu.sync_copy(agg_sh.at[pl.ds(r0, zblk)], z_v)
            pltpu.sync_copy(z_v, agg_hbm.at[cid, pl.ds(r0, zblk)])

    return k4(eu, rev0)


def _k5_body(x_ref, agg_ref, w1a_ref, w1b_ref, b1_ref, g1_ref, be1_ref,
             w2_ref, b2_ref, g2_ref, be2_ref, gn_ref, ben_ref,
             wn_ref, bn_ref, out_ref):
    n = x_ref.shape[0]
    ones = jnp.ones((1, n), jnp.float32)

    def bn(x, g, b):
        m = jnp.dot(ones, x, preferred_element_type=jnp.float32) / n
        xc = x - m
        v = jnp.dot(ones, xc * xc, preferred_element_type=jnp.float32) / n
        return g * xc * lax.rsqrt(v + EPS) + b

    agg = agg_ref[0] + agg_ref[1]
    h = (jnp.dot(x_ref[...], w1a_ref[...], preferred_element_type=jnp.float32)
         + jnp.dot(agg, w1b_ref[...], preferred_element_type=jnp.float32)
         + b1_ref[...])
    h = jnp.maximum(bn(h, g1_ref[...], be1_ref[...]), 0.0)
    h = jnp.dot(h, w2_ref[...], preferred_element_type=jnp.float32) + b2_ref[...]
    h = jnp.maximum(bn(h, g2_ref[...], be2_ref[...]), 0.0)
    h = bn(h, gn_ref[...], ben_ref[...])
    h = jnp.dot(h, wn_ref[...], preferred_element_type=jnp.float32) + bn_ref[...]
    out_ref[...] = jnp.maximum(h, 0.0)


def _node_mlp(x, aggp, w1a, w1b, b1, g1, be1, w2, b2, g2, be2, gn, ben, wn, bn):
    n, d = x.shape
    r = lambda v: v.reshape(1, d)
    return pl.pallas_call(
        _k5_body,
        out_shape=jax.ShapeDtypeStruct((n, d), jnp.float32),
    )(x, aggp, w1a, w1b, r(b1), r(g1), r(be1), w2, r(b2), r(g2), r(be2),
      r(gn), r(ben), wn, r(bn))


def kernel(node_attributes, edge_attributes, edge_indices, edge_indices_reverse,
           W_e, b_e, gamma_e, beta_e, W_g1, b_g1, gamma_g1, beta_g1,
           W_g2, b_g2, gamma_g2, beta_g2, gamma_n, beta_n, W_nu, b_nu):
    n, d = node_attributes.shape
    idx1 = edge_indices[:, 1].astype(jnp.int32)
    idx0 = edge_indices[:, 0].astype(jnp.int32)
    rev0 = edge_indices_reverse[:, 0].astype(jnp.int32)

    p, q = _make_pq(node_attributes, W_e[:d], W_e[d:2 * d])
    g = _sc_gather_sum(p, q, idx1, idx0)
    eat = edge_attributes.T
    w3 = W_e[2 * d:]
    _, _, a, c = _edge_stats(g, eat, w3, b_e, gamma_e, beta_e)
    eu = _edge_apply(g, eat, w3, b_e, a, c)
    aggp = _sc_scatter(eu, rev0, n)
    aggp = aggp[:, :n]
    node_final = _node_mlp(node_attributes, aggp,
                           W_g1[:d], W_g1[d:], b_g1, gamma_g1, beta_g1,
                           W_g2, b_g2, gamma_g2, beta_g2,
                           gamma_n, beta_n, W_nu, b_nu)
    return (node_final, eu)


# speedup vs baseline: 1.8189x; 1.1521x over previous
import functools

import numpy as np

import jax
import jax.numpy as jnp
from jax import lax
from jax.experimental import pallas as pl
from jax.experimental.pallas import tpu as pltpu
from jax.experimental.pallas import tpu_sc as plsc

EPS = 1e-3
NC = 2
NS = 16
LANES = 16


def _interleave_perm(d):
    perm = np.zeros(d, dtype=np.int32)
    for b in range(d // 32):
        for i in range(16):
            perm[32 * b + 2 * i] = 32 * b + i
            perm[32 * b + 2 * i + 1] = 32 * b + 16 + i
    return perm


def _k1_body(x_ref, w1_ref, w2_ref, p_ref, q_ref):
    x = x_ref[...]
    p_ref[...] = jnp.dot(x, w1_ref[...], preferred_element_type=jnp.float32)
    q_ref[...] = jnp.dot(x, w2_ref[...], preferred_element_type=jnp.float32)


def _make_pq(x, w1, w2):
    n, d = x.shape
    return pl.pallas_call(
        _k1_body,
        out_shape=(
            jax.ShapeDtypeStruct((n, d), jnp.float32),
            jax.ShapeDtypeStruct((n, d), jnp.float32),
        ),
    )(x, w1, w2)


def _sc_gather_sum(p, q, idx1, idx0):
    n, d = p.shape
    e = idx1.shape[0]
    nw = NC * NS
    per = e // nw
    assert per * nw == e
    chunk = 80
    nchunks = per // chunk
    assert nchunks * chunk == per
    assert nchunks % 3 == 2

    mesh = plsc.VectorSubcoreMesh(core_axis_name="c", subcore_axis_name="s")

    buf_scratch = []
    for _ in range(3):
        buf_scratch += [
            pltpu.VMEM((chunk, d), jnp.float32),
            pltpu.VMEM((chunk, d), jnp.float32),
            pltpu.SemaphoreType.DMA,
            pltpu.SemaphoreType.DMA,
            pltpu.SemaphoreType.DMA,
        ]

    @functools.partial(
        pl.kernel,
        out_type=jax.ShapeDtypeStruct((e, d), jnp.float32),
        mesh=mesh,
        scratch_types=[
            pltpu.VMEM((per,), jnp.int32),
            pltpu.VMEM((per,), jnp.int32),
        ] + buf_scratch,
    )
    def k2(p_hbm, q_hbm, i1_hbm, i0_hbm, g_hbm, i1_v, i0_v, *bufrefs):
        wid = lax.axis_index("s") * NC + lax.axis_index("c")
        base = wid * per
        bufs = [tuple(bufrefs[5 * t:5 * t + 5]) for t in range(3)]

        pltpu.sync_copy(i1_hbm.at[pl.ds(base, per)], i1_v)
        pltpu.sync_copy(i0_hbm.at[pl.ds(base, per)], i0_v)

        def issue(k, buf):
            rp, rq, sp, sq, _ = buf
            sl = pl.ds(k * chunk, chunk)
            pltpu.async_copy(p_hbm.at[i1_v.at[sl]], rp, sp)
            pltpu.async_copy(q_hbm.at[i0_v.at[sl]], rq, sq)

        def drain_write(k, buf):
            rp, _, _, _, ws = buf
            pltpu.make_async_copy(
                rp, g_hbm.at[pl.ds(base + k * chunk, chunk)], ws).wait()

        def process_add(k, buf):
            rp, rq, sp, sq, _ = buf
            sl = pl.ds(k * chunk, chunk)
            pltpu.make_async_copy(p_hbm.at[i1_v.at[sl]], rp, sp).wait()
            pltpu.make_async_copy(q_hbm.at[i0_v.at[sl]], rq, sq).wait()

            @plsc.parallel_loop(0, chunk, unroll=2)
            def _row(r):
                for j in range(d // LANES):
                    fsl = pl.ds(j * LANES, LANES)
                    rp[r, fsl] = rp[r, fsl] + rq[r, fsl]

        def write_async(k, buf):
            rp, _, _, _, ws = buf
            pltpu.async_copy(rp, g_hbm.at[pl.ds(base + k * chunk, chunk)], ws)

        issue(0, bufs[0])
        issue(1, bufs[1])

        def body3(kk, carry):
            k0 = 3 * kk
            for t in range(3):
                k = k0 + t
                b_cur = bufs[t]
                b_nxt = bufs[(t + 2) % 3]

                @pl.when(k >= 1)
                def _():
                    drain_write(k - 1, b_nxt)

                issue(k + 2, b_nxt)
                process_add(k, b_cur)
                write_async(k, b_cur)
            return carry

        lax.fori_loop(0, (nchunks - 2) // 3, body3, 0)
        for k, bi in ((nchunks - 2, 0), (nchunks - 1, 1)):
            rp = bufs[bi][0]
            process_add(k, bufs[bi])
            pltpu.sync_copy(rp, g_hbm.at[pl.ds(base + k * chunk, chunk)])
        drain_write(nchunks - 3, bufs[2])

    return k2(p, q, idx1, idx0)


def _k3a_body(nsteps, etotal, g_ref, eat_ref, w3_ref, be_ref,
              gam_ref, bet_ref, s_ref, ss_ref, a_ref, c_ref):
    i = pl.program_id(0)
    ea_term = lax.dot_general(eat_ref[...], w3_ref[...],
                              (((0,), (0,)), ((), ())),
                              preferred_element_type=jnp.float32)
    h = g_ref[...] + ea_term + be_ref[...]
    h = jnp.maximum(h, 0.0)

    @pl.when(i == 0)
    def _():
        s_ref[...] = jnp.zeros_like(s_ref)
        ss_ref[...] = jnp.zeros_like(ss_ref)

    s_ref[...] += jnp.sum(h, axis=0, keepdims=True)
    ss_ref[...] += jnp.sum(h * h, axis=0, keepdims=True)

    @pl.when(i == nsteps - 1)
    def _():
        mean = s_ref[...] / etotal
        var = ss_ref[...] / etotal - mean * mean
        a = gam_ref[...] * lax.rsqrt(var + EPS)
        a_ref[...] = a
        c_ref[...] = bet_ref[...] - mean * a


def _edge_stats(g, eat, w3, b_e, gamma_e, beta_e):
    e, d = g.shape
    de = eat.shape[0]
    blk = 6400
    nsteps = e // blk
    assert nsteps * blk == e
    body = functools.partial(_k3a_body, nsteps, float(e))
    rvec = pl.BlockSpec((1, d), lambda i: (0, 0))
    return pl.pallas_call(
        body,
        grid=(nsteps,),
        in_specs=[
            pl.BlockSpec((blk, d), lambda i: (i, 0)),
            pl.BlockSpec((de, blk), lambda i: (0, i)),
            pl.BlockSpec((de, d), lambda i: (0, 0)),
            rvec, rvec, rvec,
        ],
        out_specs=[rvec, rvec, rvec, rvec],
        out_shape=[jax.ShapeDtypeStruct((1, d), jnp.float32)] * 4,
    )(g, eat, w3, b_e.reshape(1, d), gamma_e.reshape(1, d),
      beta_e.reshape(1, d))


def _k3b_body(g_ref, eat_ref, w3_ref, be_ref, a_ref, c_ref, eu_ref):
    ea_term = lax.dot_general(eat_ref[...], w3_ref[...],
                              (((0,), (0,)), ((), ())),
                              preferred_element_type=jnp.float32)
    h = g_ref[...] + ea_term + be_ref[...]
    h = jnp.maximum(h, 0.0)
    eu_ref[...] = jnp.maximum(a_ref[...] * h + c_ref[...], 0.0)


def _edge_apply(g, eat, w3, b_e, a, c):
    e, d = g.shape
    de = eat.shape[0]
    blk = 6400
    nsteps = e // blk
    rvec = pl.BlockSpec((1, d), lambda i: (0, 0))
    return pl.pallas_call(
        _k3b_body,
        grid=(nsteps,),
        in_specs=[
            pl.BlockSpec((blk, d), lambda i: (i, 0)),
            pl.BlockSpec((de, blk), lambda i: (0, i)),
            pl.BlockSpec((de, d), lambda i: (0, 0)),
            rvec, rvec, rvec,
        ],
        out_specs=pl.BlockSpec((blk, d), lambda i: (i, 0)),
        out_shape=jax.ShapeDtypeStruct((e, d), jnp.float32),
    )(g, eat, w3, b_e.reshape(1, d), a, c)


def _sc_scatter(eu, rev0, n):
    e, d = eu.shape
    nw = NC * NS
    per = e // nw
    chunk = 80
    nchunks = per // chunk
    assert nchunks * chunk == per
    zblk = 128
    rows_per_tile = ((n + NS - 1) // NS + zblk - 1) // zblk * zblk
    npad = NS * rows_per_tile
    nz = rows_per_tile // zblk

    mesh = plsc.VectorSubcoreMesh(core_axis_name="c", subcore_axis_name="s")

    @functools.partial(
        pl.kernel,
        out_type=jax.ShapeDtypeStruct((NC, npad, d), jnp.float32),
        mesh=mesh,
        scratch_types=[
            pltpu.VMEM((zblk, d), jnp.float32),
            pltpu.VMEM_SHARED((npad, d), jnp.float32),
        ] + [
            t
            for _ in range(3)
            for t in (pltpu.VMEM((chunk,), jnp.int32),
                      pltpu.VMEM((chunk, d), jnp.float32),
                      pltpu.SemaphoreType.DMA,
                      pltpu.SemaphoreType.DMA)
        ],
    )
    def k4(eu_hbm, rev_hbm, agg_hbm, z_v, agg_sh, *bufrefs):
        cid = lax.axis_index("c")
        sid = lax.axis_index("s")
        wid = sid * NC + cid
        base = wid * per
        bufs = [tuple(bufrefs[4 * t:4 * t + 4]) for t in range(3)]

        @plsc.parallel_loop(0, zblk, unroll=2)
        def _zrow(r):
            for j in range(d // LANES):
                z_v[r, pl.ds(j * LANES, LANES)] = jnp.zeros((LANES,), jnp.float32)

        for t in range(nz):
            pltpu.sync_copy(z_v, agg_sh.at[pl.ds(sid * rows_per_tile + t * zblk, zblk)])
        plsc.subcore_barrier()

        def issue(k, buf):
            idx_v, h_v, sh, si = buf
            off = base + k * chunk
            pltpu.async_copy(eu_hbm.at[pl.ds(off, chunk)], h_v, sh)
            pltpu.async_copy(rev_hbm.at[pl.ds(off, chunk)], idx_v, si)

        def process(k, buf):
            idx_v, h_v, sh, si = buf
            off = base + k * chunk
            pltpu.make_async_copy(eu_hbm.at[pl.ds(off, chunk)], h_v, sh).wait()
            pltpu.make_async_copy(rev_hbm.at[pl.ds(off, chunk)], idx_v, si).wait()
            pltpu.sync_copy(h_v, agg_sh.at[idx_v], add=True)

        issue(0, bufs[0])
        issue(1, bufs[1])

        assert nchunks % 3 == 2
        def body3(kk, carry):
            k0 = 3 * kk
            for t in range(3):
                k = k0 + t
                issue(k + 2, bufs[(t + 2) % 3])
                process(k, bufs[t])
            return carry

        lax.fori_loop(0, (nchunks - 2) // 3, body3, 0)
        process(nchunks - 2, bufs[0])
        process(nchunks - 1, bufs[1])
        plsc.subcore_barrier()

        for t in range(nz):
            r0 = sid * rows_per_tile + t * zblk
            pltpu.sync_copy(agg_sh.at[pl.ds(r0, zblk)], z_v)
            pltpu.sync_copy(z_v, agg_hbm.at[cid, pl.ds(r0, zblk)])

    return k4(eu, rev0)


def _k5_body(x_ref, agg_ref, w1a_ref, w1b_ref, b1_ref, g1_ref, be1_ref,
             w2_ref, b2_ref, g2_ref, be2_ref, gn_ref, ben_ref,
             wn_ref, bn_ref, out_ref):
    n = x_ref.shape[0]
    ones = jnp.ones((1, n), jnp.float32)

    def bn(x, g, b):
        m = jnp.dot(ones, x, preferred_element_type=jnp.float32) / n
        xc = x - m
        v = jnp.dot(ones, xc * xc, preferred_element_type=jnp.float32) / n
        return g * xc * lax.rsqrt(v + EPS) + b

    agg = agg_ref[0] + agg_ref[1]
    h = (jnp.dot(x_ref[...], w1a_ref[...], preferred_element_type=jnp.float32)
         + jnp.dot(agg, w1b_ref[...], preferred_element_type=jnp.float32)
         + b1_ref[...])
    h = jnp.maximum(bn(h, g1_ref[...], be1_ref[...]), 0.0)
    h = jnp.dot(h, w2_ref[...], preferred_element_type=jnp.float32) + b2_ref[...]
    h = jnp.maximum(bn(h, g2_ref[...], be2_ref[...]), 0.0)
    h = bn(h, gn_ref[...], ben_ref[...])
    h = jnp.dot(h, wn_ref[...], preferred_element_type=jnp.float32) + bn_ref[...]
    out_ref[...] = jnp.maximum(h, 0.0)


def _node_mlp(x, aggp, w1a, w1b, b1, g1, be1, w2, b2, g2, be2, gn, ben, wn, bn):
    n, d = x.shape
    r = lambda v: v.reshape(1, d)
    return pl.pallas_call(
        _k5_body,
        out_shape=jax.ShapeDtypeStruct((n, d), jnp.float32),
    )(x, aggp, w1a, w1b, r(b1), r(g1), r(be1), w2, r(b2), r(g2), r(be2),
      r(gn), r(ben), wn, r(bn))


def kernel(node_attributes, edge_attributes, edge_indices, edge_indices_reverse,
           W_e, b_e, gamma_e, beta_e, W_g1, b_g1, gamma_g1, beta_g1,
           W_g2, b_g2, gamma_g2, beta_g2, gamma_n, beta_n, W_nu, b_nu):
    n, d = node_attributes.shape
    idx1 = edge_indices[:, 1].astype(jnp.int32)
    idx0 = edge_indices[:, 0].astype(jnp.int32)
    rev0 = edge_indices_reverse[:, 0].astype(jnp.int32)

    p, q = _make_pq(node_attributes, W_e[:d], W_e[d:2 * d])
    g = _sc_gather_sum(p, q, idx1, idx0)
    eat = edge_attributes.T
    w3 = W_e[2 * d:]
    _, _, a, c = _edge_stats(g, eat, w3, b_e, gamma_e, beta_e)
    eu = _edge_apply(g, eat, w3, b_e, a, c)
    aggp = _sc_scatter(eu, rev0, n)
    aggp = aggp[:, :n]
    node_final = _node_mlp(node_attributes, aggp,
                           W_g1[:d], W_g1[d:], b_g1, gamma_g1, beta_g1,
                           W_g2, b_g2, gamma_g2, beta_g2,
                           gamma_n, beta_n, W_nu, b_nu)
    return (node_final, eu)
